# Initial kernel scaffold; baseline (speedup 1.0000x reference)
import jax
import jax.numpy as jnp
from jax.experimental import pallas as pl


def kernel(x):
    raise NotImplementedError("write your pallas kernel here")


# speedup vs baseline: 1.6925x; 1.6925x over previous
import functools

import jax
import jax.numpy as jnp
from jax import lax
from jax.experimental import pallas as pl
from jax.experimental.pallas import tpu as pltpu
from jax.experimental.pallas import tpu_sc as plsc

N_POINTS = 1048576
NUM_WORKERS = 32
PTS_PER_W = N_POINTS // NUM_WORKERS
FLT_PER_W = PTS_PER_W * 2
UNROLL = 4

_CONST = 1.3862943611198906 - 1.8378770664093453 - 2.772588722239781

_LN2 = 0.6931471805599453
_SQRT2 = 1.4142135623730951
_LOG_POLY = (
    7.0376836292e-2, -1.1514610310e-1, 1.1676998740e-1, -1.2420140846e-1,
    1.4249322787e-1, -1.6668057665e-1, 2.0000714765e-1, -2.4999993993e-1,
    3.3333331174e-1,
)


def _fast_log(s):
    bits = lax.bitcast_convert_type(s, jnp.int32)
    e = lax.shift_right_logical(bits, 23) - 127
    m = lax.bitcast_convert_type(
        jnp.bitwise_or(jnp.bitwise_and(bits, 0x007FFFFF), 0x3F800000),
        jnp.float32)
    big = m > jnp.float32(_SQRT2)
    m = jnp.where(big, m * jnp.float32(0.5), m)
    ef = e.astype(jnp.float32) + jnp.where(big, jnp.float32(1.0),
                                           jnp.float32(0.0))
    z = m - jnp.float32(1.0)
    r = jnp.float32(_LOG_POLY[0])
    for c in _LOG_POLY[1:]:
        r = r * z + jnp.float32(c)
    z2 = z * z
    y = z * z2 * r - jnp.float32(0.5) * z2 + z
    return y + ef * jnp.float32(_LN2)


def _axis_term(v):
    u = v * jnp.float32(4.0)
    s = (jnp.float32(1.0)
         + jnp.exp(u - jnp.float32(2.0))
         + jnp.exp(u + u - jnp.float32(8.0))
         + jnp.exp(u * jnp.float32(3.0) - jnp.float32(18.0)))
    return _fast_log(s) - jnp.float32(2.0) * v * v


_MESH = plsc.VectorSubcoreMesh(core_axis_name="c", subcore_axis_name="s")


@functools.partial(
    pl.kernel,
    mesh=_MESH,
    out_type=jax.ShapeDtypeStruct((N_POINTS,), jnp.float32),
    scratch_types=[
        pltpu.VMEM((PTS_PER_W,), jnp.float32),
        pltpu.VMEM((PTS_PER_W,), jnp.float32),
        pltpu.VMEM((PTS_PER_W,), jnp.float32),
    ],
)
def _gmm_logpdf(x_hbm, out_hbm, xv0, xv1, ov):
    wid = lax.axis_index("s") * 2 + lax.axis_index("c")
    base = wid * PTS_PER_W
    pltpu.sync_copy(x_hbm.at[pl.ds(base, PTS_PER_W)], xv0)
    pltpu.sync_copy(x_hbm.at[pl.ds(N_POINTS + base, PTS_PER_W)], xv1)

    def body(i, carry):
        for j in range(UNROLL):
            o = (i * UNROLL + j) * 16
            x0 = xv0[pl.ds(o, 16)]
            x1 = xv1[pl.ds(o, 16)]
            ov[pl.ds(o, 16)] = (_axis_term(x0) + _axis_term(x1)
                                + jnp.float32(_CONST))
        return carry

    lax.fori_loop(0, PTS_PER_W // 16 // UNROLL, body, 0)
    pltpu.sync_copy(ov, out_hbm.at[pl.ds(base, PTS_PER_W)])


def kernel(x):
    return _gmm_logpdf(x.T.reshape(-1))
